# baseline (speedup 1.0000x reference)
import functools

import jax
import jax.numpy as jnp
from jax import lax
from jax.experimental import pallas as pl
from jax.experimental.pallas import tpu as pltpu
from jax.experimental.pallas import tpu_sc as plsc

B, H, W, K = 4, 512, 512, 8
P, C = 1000000, 3
N = B * H * W
PACK = 4
D = 8 * PACK
TR = (P + 16) // PACK

_info = plsc.get_sparse_core_info()
NC, NS, L = _info.num_cores, _info.num_subcores, _info.num_lanes
NW = NC * NS
PER_W = N // NW
S = 2048
NCHUNK = PER_W // S
SROW = S // 128

_mesh = plsc.VectorSubcoreMesh(core_axis_name="c", subcore_axis_name="s")


@functools.partial(
    pl.kernel,
    mesh=_mesh,
    out_type=jax.ShapeDtypeStruct((N, D), jnp.float32),
    scratch_types=[
        pltpu.VMEM((SROW, 128), jnp.int32),
        pltpu.VMEM((S,), jnp.int32),
        pltpu.VMEM((S, D), jnp.float32),
        pltpu.SemaphoreType.DMA,
    ],
    compiler_params=pltpu.CompilerParams(use_tc_tiling_on_sc=False),
)
def _render(idx_hbm, table_hbm, out_hbm, idx_v, sidx_v, rows_v, sem):
    wid = lax.axis_index("s") * NC + lax.axis_index("c")
    base = wid * PER_W
    bg_row = jnp.full((L,), P, jnp.int32)

    for ch in range(NCHUNK):
        off = base + ch * S
        pltpu.sync_copy(idx_hbm.at[pl.ds(off // 128, SROW)], idx_v)

        def remap_body(r, _):
            for c in range(8):
                v = idx_v[r, pl.ds(c * L, L)]
                sidx_v[pl.ds(r * 128 + c * L, L)] = (
                    jnp.where(v < 0, bg_row, v) >> 2)
            return 0

        lax.fori_loop(0, SROW, remap_body, 0)

        pltpu.async_copy(table_hbm.at[sidx_v], rows_v, sem).wait()

        pltpu.sync_copy(rows_v, out_hbm.at[pl.ds(off, S)])


def kernel(fragments_idx, features_packed):
    sel0 = jnp.zeros((K,), jnp.int32).at[0].set(1)
    idx0 = jnp.sum(fragments_idx.reshape(N, K) * sel0, axis=-1)

    table = jnp.pad(
        jnp.concatenate(
            [features_packed, jnp.ones((16, C), jnp.float32)], axis=0),
        ((0, 0), (0, 5))).reshape(TR, D)

    out32 = _render(idx0.reshape(N // 128, 128), table)

    q = jnp.where(idx0 < 0, 0, idx0 & 3)[:, None]
    out = jnp.where(
        q < 2,
        jnp.where(q == 0, out32[:, 0:C], out32[:, 8:8 + C]),
        jnp.where(q == 2, out32[:, 16:16 + C], out32[:, 24:24 + C]))
    return out.reshape(B, H, W, C)


# speedup vs baseline: 1.3804x; 1.3804x over previous
import functools

import jax
import jax.numpy as jnp
from jax import lax
from jax.experimental import pallas as pl
from jax.experimental.pallas import tpu as pltpu
from jax.experimental.pallas import tpu_sc as plsc

B, H, W, K = 4, 512, 512, 8
P, C = 1000000, 3
N = B * H * W
D = 8

_info = plsc.get_sparse_core_info()
NC, NS, L = _info.num_cores, _info.num_subcores, _info.num_lanes
NW = NC * NS
PER_W = N // NW
S = 8192
NCHUNK = PER_W // S

_mesh = plsc.VectorSubcoreMesh(core_axis_name="c", subcore_axis_name="s")


@functools.partial(
    pl.kernel,
    mesh=_mesh,
    out_type=jax.ShapeDtypeStruct((N, D), jnp.float32),
    scratch_types=[
        pltpu.VMEM((S,), jnp.int32),
        pltpu.VMEM((S,), jnp.int32),
        pltpu.VMEM((S, D), jnp.float32),
        pltpu.SemaphoreType.DMA,
    ],
    compiler_params=pltpu.CompilerParams(use_tc_tiling_on_sc=False),
)
def _render(idx_hbm, table_hbm, out_hbm, idx_v, sidx_v, rows_v, sem):
    wid = lax.axis_index("s") * NC + lax.axis_index("c")
    base = wid * PER_W
    bg_row = jnp.full((L,), P, jnp.int32)

    for ch in range(NCHUNK):
        off = base + ch * S
        pltpu.sync_copy(idx_hbm.at[pl.ds(off, S)], idx_v)

        def remap_body(i, _):
            v = idx_v[pl.ds(i * L, L)]
            sidx_v[pl.ds(i * L, L)] = jnp.where(v < 0, bg_row, v)
            return 0

        lax.fori_loop(0, S // L, remap_body, 0)

        pltpu.async_copy(table_hbm.at[sidx_v], rows_v, sem).wait()

        pltpu.sync_copy(rows_v, out_hbm.at[pl.ds(off, S)])


def kernel(fragments_idx, features_packed):
    sel0 = jnp.zeros((K,), jnp.int32).at[0].set(1)
    idx0 = jnp.sum(fragments_idx * sel0, axis=-1).reshape(N)

    table = jnp.pad(
        jnp.concatenate(
            [features_packed, jnp.ones((8, C), jnp.float32)], axis=0),
        ((0, 0), (0, D - C)))

    out8 = _render(idx0, table)

    sel = jnp.zeros((D, C), jnp.float32).at[0, 0].set(1.0)
    sel = sel.at[1, 1].set(1.0).at[2, 2].set(1.0)
    out = out8 @ sel
    return out.reshape(B, H, W, C)
